# baseline (speedup 1.0000x reference)
import functools

import jax
import jax.numpy as jnp
from jax import lax
from jax.experimental import pallas as pl
from jax.experimental.pallas import tpu as pltpu
from jax.experimental.pallas import tpu_sc as plsc

NC = 2
NS = 16
NW = NC * NS
C = 128


def _make_deg(K, NP, RPT):
    DW = 16
    mesh = plsc.VectorSubcoreMesh(core_axis_name="c", subcore_axis_name="s")

    @functools.partial(
        pl.kernel,
        out_type=jax.ShapeDtypeStruct((NC, NP, DW), jnp.float32),
        mesh=mesh,
        compiler_params=pltpu.CompilerParams(use_tc_tiling_on_sc=False),
        scratch_types=[
            pltpu.VMEM((K, C), jnp.int32),
            pltpu.VMEM((C, DW), jnp.float32),
            pltpu.VMEM_SHARED((NP, DW), jnp.float32),
        ],
    )
    def deg(dst_hbm, z_hbm, out_hbm, dst_v, ones_v, acc):
        cid = lax.axis_index("c")
        sid = lax.axis_index("s")
        wid = sid * NC + cid
        pltpu.sync_copy(dst_hbm.at[wid], dst_v)
        for r in range(C):
            ones_v[r] = jnp.full((DW,), 1.0, jnp.float32)
        pltpu.sync_copy(z_hbm, acc.at[pl.ds(sid * RPT, RPT)])
        plsc.subcore_barrier()

        def body(j, carry):
            pltpu.sync_copy(ones_v, acc.at[dst_v.at[j]], add=True)
            return carry

        lax.fori_loop(0, K, body, 0)
        plsc.subcore_barrier()
        pltpu.sync_copy(acc.at[pl.ds(sid * RPT, RPT)],
                        out_hbm.at[cid, pl.ds(sid * RPT, RPT)])

    return deg


def _make_agg(D, K, NP, RPT):
    mesh = plsc.VectorSubcoreMesh(core_axis_name="c", subcore_axis_name="s")

    @functools.partial(
        pl.kernel,
        out_type=jax.ShapeDtypeStruct((NC, NP, D), jnp.float32),
        mesh=mesh,
        compiler_params=pltpu.CompilerParams(use_tc_tiling_on_sc=False),
        scratch_types=[
            pltpu.VMEM((K, C), jnp.int32),
            pltpu.VMEM((K, C), jnp.int32),
            pltpu.VMEM((C, D), jnp.float32),
            pltpu.VMEM((C, D), jnp.float32),
            pltpu.VMEM_SHARED((NP, D), jnp.float32),
            pltpu.SemaphoreType.DMA,
            pltpu.SemaphoreType.DMA,
        ],
    )
    def agg(g_hbm, src_hbm, dst_hbm, z_hbm, out_hbm,
            src_v, dst_v, buf0, buf1, acc, sem0, sem1):
        cid = lax.axis_index("c")
        sid = lax.axis_index("s")
        wid = sid * NC + cid
        pltpu.sync_copy(src_hbm.at[wid], src_v)
        pltpu.sync_copy(dst_hbm.at[wid], dst_v)
        pltpu.sync_copy(z_hbm, acc.at[pl.ds(sid * RPT, RPT)])
        plsc.subcore_barrier()

        def body(j, carry):
            pltpu.sync_copy(g_hbm.at[src_v.at[j]], buf0)
            pltpu.sync_copy(buf0, acc.at[dst_v.at[j]], add=True)
            return carry

        lax.fori_loop(0, K, body, 0)
        plsc.subcore_barrier()
        pltpu.sync_copy(acc.at[pl.ds(sid * RPT, RPT)],
                        out_hbm.at[cid, pl.ds(sid * RPT, RPT)])

    return agg


def _mm_scale_body(x_ref, w_ref, d_ref, o_ref):
    o_ref[...] = jnp.dot(x_ref[...], w_ref[...],
                         preferred_element_type=jnp.float32) * d_ref[...]


def _mm_scale(x, w, d, R):
    N, DIN = x.shape
    DH = w.shape[1]
    return pl.pallas_call(
        _mm_scale_body,
        grid=(pl.cdiv(N, R),),
        in_specs=[
            pl.BlockSpec((R, DIN), lambda i: (i, 0)),
            pl.BlockSpec((DIN, DH), lambda i: (0, 0)),
            pl.BlockSpec((R, 1), lambda i: (i, 0)),
        ],
        out_specs=pl.BlockSpec((R, DH), lambda i: (i, 0)),
        out_shape=jax.ShapeDtypeStruct((N, DH), jnp.float32),
    )(x, w, d)


def _fuse1_body(pa_ref, pb_ref, g1_ref, d_ref, b1_ref, w2_ref, o_ref):
    s = pa_ref[...] + pb_ref[...] + g1_ref[...]
    h = jnp.maximum(d_ref[...] * s + b1_ref[...], 0.0)
    o_ref[...] = jnp.dot(h, w2_ref[...],
                         preferred_element_type=jnp.float32) * d_ref[...]


def _fuse1(pa, pb, g1, d, b1, w2, R):
    N, DH = g1.shape
    DP = w2.shape[1]
    return pl.pallas_call(
        _fuse1_body,
        grid=(pl.cdiv(N, R),),
        in_specs=[
            pl.BlockSpec((R, DH), lambda i: (i, 0)),
            pl.BlockSpec((R, DH), lambda i: (i, 0)),
            pl.BlockSpec((R, DH), lambda i: (i, 0)),
            pl.BlockSpec((R, 1), lambda i: (i, 0)),
            pl.BlockSpec((1, DH), lambda i: (0, 0)),
            pl.BlockSpec((DH, DP), lambda i: (0, 0)),
        ],
        out_specs=pl.BlockSpec((R, DP), lambda i: (i, 0)),
        out_shape=jax.ShapeDtypeStruct((N, DP), jnp.float32),
    )(pa, pb, g1, d, b1, w2)


def _fuse2_body(pa_ref, pb_ref, g2_ref, d_ref, b2_ref, o_ref, *, DOUT):
    s = d_ref[...] * (pa_ref[...] + pb_ref[...] + g2_ref[...]) + b2_ref[...]
    s = s[:, :DOUT]
    m = jnp.max(s, axis=1, keepdims=True)
    e = jnp.exp(s - m)
    o_ref[...] = e / jnp.sum(e, axis=1, keepdims=True)


def _fuse2(pa, pb, g2, d, b2, DOUT, R):
    N, DP = g2.shape
    return pl.pallas_call(
        functools.partial(_fuse2_body, DOUT=DOUT),
        grid=(pl.cdiv(N, R),),
        in_specs=[
            pl.BlockSpec((R, DP), lambda i: (i, 0)),
            pl.BlockSpec((R, DP), lambda i: (i, 0)),
            pl.BlockSpec((R, DP), lambda i: (i, 0)),
            pl.BlockSpec((R, 1), lambda i: (i, 0)),
            pl.BlockSpec((1, DP), lambda i: (0, 0)),
        ],
        out_specs=pl.BlockSpec((R, DOUT), lambda i: (i, 0)),
        out_shape=jax.ShapeDtypeStruct((N, DOUT), jnp.float32),
    )(pa, pb, g2, d, b2)


def kernel(x, edge_index, W1, b1, W2, b2):
    N, DIN = x.shape
    DH = W1.shape[1]
    DOUT = W2.shape[1]
    DP = 48
    E = edge_index.shape[1]

    K = -(-E // (NW * C))
    if K % 2:
        K += 1
    pad = NW * K * C - E

    NP = (NS * 8) * (-(-(N + 1) // (NS * 8)))
    RPT = NP // NS

    dump = N + jnp.arange(pad, dtype=jnp.int32) % jnp.int32(NP - N)
    src = jnp.concatenate(
        [edge_index[0], jnp.zeros((pad,), jnp.int32)]).reshape(NW, K, C)
    dst = jnp.concatenate(
        [edge_index[1], dump]).reshape(NW, K, C)

    degp = _make_deg(K, NP, RPT)(dst, jnp.zeros((RPT, 16), jnp.float32))
    deg = degp[0, :N, 0] + degp[1, :N, 0] + 1.0
    dcol = (deg ** -0.5).reshape(N, 1)

    R = 1000 if N % 1000 == 0 else 8 * (-(-N // 80) // 8 * 8 or 8)

    g1 = _mm_scale(x, W1, dcol, R)
    p1 = _make_agg(DH, K, NP, RPT)(
        g1, src, dst, jnp.zeros((RPT, DH), jnp.float32))

    W2p = jnp.pad(W2, ((0, 0), (0, DP - DOUT)))
    b2p = jnp.pad(b2, (0, DP - DOUT)).reshape(1, DP)
    g2 = _fuse1(p1[0, :N], p1[1, :N], g1, dcol, b1.reshape(1, DH), W2p, R)
    p2 = _make_agg(DP, K, NP, RPT)(
        g2, src, dst, jnp.zeros((RPT, DP), jnp.float32))

    return _fuse2(p2[0, :N], p2[1, :N], g2, dcol, b2p, DOUT, R)


# speedup vs baseline: 1.8996x; 1.8996x over previous
import functools

import jax
import jax.numpy as jnp
from jax import lax
from jax.experimental import pallas as pl
from jax.experimental.pallas import tpu as pltpu
from jax.experimental.pallas import tpu_sc as plsc

NC = 2
NS = 16
NW = NC * NS
C = 128


def _make_deg(K, NP, RPT):
    DW = 16
    mesh = plsc.VectorSubcoreMesh(core_axis_name="c", subcore_axis_name="s")

    @functools.partial(
        pl.kernel,
        out_type=jax.ShapeDtypeStruct((NC, NP, DW), jnp.float32),
        mesh=mesh,
        compiler_params=pltpu.CompilerParams(use_tc_tiling_on_sc=False),
        scratch_types=[
            pltpu.VMEM((K, C), jnp.int32),
            pltpu.VMEM((C, DW), jnp.float32),
            pltpu.VMEM_SHARED((NP, DW), jnp.float32),
        ],
    )
    def deg(dst_hbm, z_hbm, out_hbm, dst_v, ones_v, acc):
        cid = lax.axis_index("c")
        sid = lax.axis_index("s")
        wid = sid * NC + cid
        pltpu.sync_copy(dst_hbm.at[wid], dst_v)
        for r in range(C):
            ones_v[r] = jnp.full((DW,), 1.0, jnp.float32)
        pltpu.sync_copy(z_hbm, acc.at[pl.ds(sid * RPT, RPT)])
        plsc.subcore_barrier()

        def body(j, carry):
            pltpu.sync_copy(ones_v, acc.at[dst_v.at[j]], add=True)
            return carry

        lax.fori_loop(0, K, body, 0)
        plsc.subcore_barrier()
        pltpu.sync_copy(acc.at[pl.ds(sid * RPT, RPT)],
                        out_hbm.at[cid, pl.ds(sid * RPT, RPT)])

    return deg


def _make_agg(D, K, NP, RPT):
    mesh = plsc.VectorSubcoreMesh(core_axis_name="c", subcore_axis_name="s")

    @functools.partial(
        pl.kernel,
        out_type=jax.ShapeDtypeStruct((NC, NP, D), jnp.float32),
        mesh=mesh,
        compiler_params=pltpu.CompilerParams(use_tc_tiling_on_sc=False),
        scratch_types=[
            pltpu.VMEM((K, C), jnp.int32),
            pltpu.VMEM((K, C), jnp.int32),
            pltpu.VMEM((C, D), jnp.float32),
            pltpu.VMEM((C, D), jnp.float32),
            pltpu.VMEM_SHARED((NP, D), jnp.float32),
            pltpu.SemaphoreType.DMA,
            pltpu.SemaphoreType.DMA,
        ],
    )
    def agg(g_hbm, src_hbm, dst_hbm, z_hbm, out_hbm,
            src_v, dst_v, buf0, buf1, acc, sem0, sem1):
        cid = lax.axis_index("c")
        sid = lax.axis_index("s")
        wid = sid * NC + cid
        pltpu.sync_copy(src_hbm.at[wid], src_v)
        pltpu.sync_copy(dst_hbm.at[wid], dst_v)
        pltpu.sync_copy(z_hbm, acc.at[pl.ds(sid * RPT, RPT)])
        plsc.subcore_barrier()

        def body(j, carry):
            pltpu.sync_copy(g_hbm.at[src_v.at[j]], buf0)
            pltpu.sync_copy(buf0, acc.at[dst_v.at[j]], add=True)
            return carry

        lax.fori_loop(0, K, body, 0)
        plsc.subcore_barrier()
        pltpu.sync_copy(acc.at[pl.ds(sid * RPT, RPT)],
                        out_hbm.at[cid, pl.ds(sid * RPT, RPT)])

    return agg


def _mm_scale_body(x_ref, w_ref, d_ref, o_ref):
    o_ref[...] = jnp.dot(x_ref[...], w_ref[...],
                         preferred_element_type=jnp.float32) * d_ref[...]


def _mm_scale(x, w, d, R):
    N, DIN = x.shape
    DH = w.shape[1]
    return pl.pallas_call(
        _mm_scale_body,
        grid=(pl.cdiv(N, R),),
        in_specs=[
            pl.BlockSpec((R, DIN), lambda i: (i, 0)),
            pl.BlockSpec((DIN, DH), lambda i: (0, 0)),
            pl.BlockSpec((R, 1), lambda i: (i, 0)),
        ],
        out_specs=pl.BlockSpec((R, DH), lambda i: (i, 0)),
        out_shape=jax.ShapeDtypeStruct((N, DH), jnp.float32),
    )(x, w, d)


def _fuse1_body(pa_ref, pb_ref, g1_ref, d_ref, b1_ref, w2_ref, o_ref):
    s = pa_ref[...] + pb_ref[...] + g1_ref[...]
    h = jnp.maximum(d_ref[...] * s + b1_ref[...], 0.0)
    o_ref[...] = jnp.dot(h, w2_ref[...],
                         preferred_element_type=jnp.float32) * d_ref[...]


def _fuse1(pa, pb, g1, d, b1, w2, R):
    N, DH = g1.shape
    DP = w2.shape[1]
    return pl.pallas_call(
        _fuse1_body,
        grid=(pl.cdiv(N, R),),
        in_specs=[
            pl.BlockSpec((R, DH), lambda i: (i, 0)),
            pl.BlockSpec((R, DH), lambda i: (i, 0)),
            pl.BlockSpec((R, DH), lambda i: (i, 0)),
            pl.BlockSpec((R, 1), lambda i: (i, 0)),
            pl.BlockSpec((1, DH), lambda i: (0, 0)),
            pl.BlockSpec((DH, DP), lambda i: (0, 0)),
        ],
        out_specs=pl.BlockSpec((R, DP), lambda i: (i, 0)),
        out_shape=jax.ShapeDtypeStruct((N, DP), jnp.float32),
    )(pa, pb, g1, d, b1, w2)


def _fuse2_body(pa_ref, pb_ref, g2_ref, d_ref, b2_ref, o_ref, *, DOUT):
    s = d_ref[...] * (pa_ref[...] + pb_ref[...] + g2_ref[...]) + b2_ref[...]
    s = s[:, :DOUT]
    m = jnp.max(s, axis=1, keepdims=True)
    e = jnp.exp(s - m)
    o_ref[...] = e / jnp.sum(e, axis=1, keepdims=True)


def _fuse2(pa, pb, g2, d, b2, DOUT, R):
    N, DP = g2.shape
    return pl.pallas_call(
        functools.partial(_fuse2_body, DOUT=DOUT),
        grid=(pl.cdiv(N, R),),
        in_specs=[
            pl.BlockSpec((R, DP), lambda i: (i, 0)),
            pl.BlockSpec((R, DP), lambda i: (i, 0)),
            pl.BlockSpec((R, DP), lambda i: (i, 0)),
            pl.BlockSpec((R, 1), lambda i: (i, 0)),
            pl.BlockSpec((1, DP), lambda i: (0, 0)),
        ],
        out_specs=pl.BlockSpec((R, DOUT), lambda i: (i, 0)),
        out_shape=jax.ShapeDtypeStruct((N, DOUT), jnp.float32),
    )(pa, pb, g2, d, b2)


def kernel(x, edge_index, W1, b1, W2, b2):
    N, DIN = x.shape
    DH = W1.shape[1]
    DOUT = W2.shape[1]
    DP = 48
    E = edge_index.shape[1]

    K = -(-E // (NW * C))
    if K % 2:
        K += 1
    pad = NW * K * C - E

    NP = (NS * 8) * (-(-(N + 1) // (NS * 8)))
    RPT = NP // NS

    dump = N + jnp.arange(pad, dtype=jnp.int32) % jnp.int32(NP - N)
    psrc = jnp.arange(pad, dtype=jnp.int32) % jnp.int32(N)
    src = jnp.concatenate(
        [edge_index[0], psrc]).reshape(NW, K, C)
    dst = jnp.concatenate(
        [edge_index[1], dump]).reshape(NW, K, C)

    degp = _make_deg(K, NP, RPT)(dst, jnp.zeros((RPT, 16), jnp.float32))
    deg = degp[0, :N, 0] + degp[1, :N, 0] + 1.0
    dcol = (deg ** -0.5).reshape(N, 1)

    R = 1000 if N % 1000 == 0 else 8 * (-(-N // 80) // 8 * 8 or 8)

    g1 = _mm_scale(x, W1, dcol, R)
    p1 = _make_agg(DH, K, NP, RPT)(
        g1, src, dst, jnp.zeros((RPT, DH), jnp.float32))

    W2p = jnp.pad(W2, ((0, 0), (0, DP - DOUT)))
    b2p = jnp.pad(b2, (0, DP - DOUT)).reshape(1, DP)
    g2 = _fuse1(p1[0, :N], p1[1, :N], g1, dcol, b1.reshape(1, DH), W2p, R)
    p2 = _make_agg(DP, K, NP, RPT)(
        g2, src, dst, jnp.zeros((RPT, DP), jnp.float32))

    return _fuse2(p2[0, :N], p2[1, :N], g2, dcol, b2p, DOUT, R)


# speedup vs baseline: 2.8826x; 1.5175x over previous
import functools

import jax
import jax.numpy as jnp
from jax import lax
from jax.experimental import pallas as pl
from jax.experimental.pallas import tpu as pltpu
from jax.experimental.pallas import tpu_sc as plsc

NC = 2
NS = 16
NW = NC * NS
C = 128


def _make_deg(K, NP, RPT):
    DW = 16
    mesh = plsc.VectorSubcoreMesh(core_axis_name="c", subcore_axis_name="s")

    @functools.partial(
        pl.kernel,
        out_type=jax.ShapeDtypeStruct((NC, NP, DW), jnp.float32),
        mesh=mesh,
        compiler_params=pltpu.CompilerParams(use_tc_tiling_on_sc=False),
        scratch_types=[
            pltpu.VMEM((K, C), jnp.int32),
            pltpu.VMEM((C, DW), jnp.float32),
            pltpu.VMEM_SHARED((NP, DW), jnp.float32),
        ],
    )
    def deg(dst_hbm, z_hbm, out_hbm, dst_v, ones_v, acc):
        cid = lax.axis_index("c")
        sid = lax.axis_index("s")
        wid = sid * NC + cid
        pltpu.sync_copy(dst_hbm.at[wid], dst_v)
        for r in range(C):
            ones_v[r] = jnp.full((DW,), 1.0, jnp.float32)
        pltpu.sync_copy(z_hbm, acc.at[pl.ds(sid * RPT, RPT)])
        plsc.subcore_barrier()

        def body(j, carry):
            pltpu.sync_copy(ones_v, acc.at[dst_v.at[j]], add=True)
            return carry

        lax.fori_loop(0, K, body, 0)
        plsc.subcore_barrier()
        pltpu.sync_copy(acc.at[pl.ds(sid * RPT, RPT)],
                        out_hbm.at[cid, pl.ds(sid * RPT, RPT)])

    return deg


def _make_agg(D, K, NP, RPT):
    mesh = plsc.VectorSubcoreMesh(core_axis_name="c", subcore_axis_name="s")

    @functools.partial(
        pl.kernel,
        out_type=jax.ShapeDtypeStruct((NC, NP, D), jnp.float32),
        mesh=mesh,
        compiler_params=pltpu.CompilerParams(use_tc_tiling_on_sc=False),
        scratch_types=[
            pltpu.VMEM((K, C), jnp.int32),
            pltpu.VMEM((K, C), jnp.int32),
            pltpu.VMEM((4, C, D), jnp.float32),
            pltpu.VMEM_SHARED((NP, D), jnp.float32),
            pltpu.SemaphoreType.DMA,
            pltpu.SemaphoreType.DMA,
            pltpu.SemaphoreType.DMA,
            pltpu.SemaphoreType.DMA,
        ],
    )
    def agg(g_hbm, src_hbm, dst_hbm, z_hbm, out_hbm,
            src_v, dst_v, bufs, acc, s0, s1, s2, s3):
        sems = (s0, s1, s2, s3)
        cid = lax.axis_index("c")
        sid = lax.axis_index("s")
        wid = sid * NC + cid
        pltpu.sync_copy(src_hbm.at[wid], src_v)
        pltpu.sync_copy(dst_hbm.at[wid], dst_v)
        pltpu.sync_copy(z_hbm, acc.at[pl.ds(sid * RPT, RPT)])
        for b in range(4):
            pltpu.async_copy(g_hbm.at[src_v.at[b]], bufs.at[b], sems[b])
        plsc.subcore_barrier()

        def body(jj, carry):
            for b in range(4):
                j = 4 * jj + b
                pltpu.make_async_copy(
                    g_hbm.at[src_v.at[j]], bufs.at[b], sems[b]).wait()
                pltpu.sync_copy(bufs.at[b], acc.at[dst_v.at[j]], add=True)
                pltpu.async_copy(
                    g_hbm.at[src_v.at[jnp.minimum(j + 4, K - 1)]],
                    bufs.at[b], sems[b])
            return carry

        lax.fori_loop(0, K // 4, body, 0)
        for b in range(4):
            pltpu.make_async_copy(
                g_hbm.at[src_v.at[0]], bufs.at[b], sems[b]).wait()
        plsc.subcore_barrier()
        pltpu.sync_copy(acc.at[pl.ds(sid * RPT, RPT)],
                        out_hbm.at[cid, pl.ds(sid * RPT, RPT)])

    return agg


def _mm_scale_body(x_ref, w_ref, d_ref, o_ref):
    o_ref[...] = jnp.dot(x_ref[...], w_ref[...],
                         preferred_element_type=jnp.float32) * d_ref[...]


def _mm_scale(x, w, d, R):
    N, DIN = x.shape
    DH = w.shape[1]
    return pl.pallas_call(
        _mm_scale_body,
        grid=(pl.cdiv(N, R),),
        in_specs=[
            pl.BlockSpec((R, DIN), lambda i: (i, 0)),
            pl.BlockSpec((DIN, DH), lambda i: (0, 0)),
            pl.BlockSpec((R, 1), lambda i: (i, 0)),
        ],
        out_specs=pl.BlockSpec((R, DH), lambda i: (i, 0)),
        out_shape=jax.ShapeDtypeStruct((N, DH), jnp.float32),
    )(x, w, d)


def _fuse1_body(pa_ref, pb_ref, g1_ref, d_ref, b1_ref, w2_ref, o_ref):
    s = pa_ref[...] + pb_ref[...] + g1_ref[...]
    h = jnp.maximum(d_ref[...] * s + b1_ref[...], 0.0)
    o_ref[...] = jnp.dot(h, w2_ref[...],
                         preferred_element_type=jnp.float32) * d_ref[...]


def _fuse1(pa, pb, g1, d, b1, w2, R):
    N, DH = g1.shape
    DP = w2.shape[1]
    return pl.pallas_call(
        _fuse1_body,
        grid=(pl.cdiv(N, R),),
        in_specs=[
            pl.BlockSpec((R, DH), lambda i: (i, 0)),
            pl.BlockSpec((R, DH), lambda i: (i, 0)),
            pl.BlockSpec((R, DH), lambda i: (i, 0)),
            pl.BlockSpec((R, 1), lambda i: (i, 0)),
            pl.BlockSpec((1, DH), lambda i: (0, 0)),
            pl.BlockSpec((DH, DP), lambda i: (0, 0)),
        ],
        out_specs=pl.BlockSpec((R, DP), lambda i: (i, 0)),
        out_shape=jax.ShapeDtypeStruct((N, DP), jnp.float32),
    )(pa, pb, g1, d, b1, w2)


def _fuse2_body(pa_ref, pb_ref, g2_ref, d_ref, b2_ref, o_ref, *, DOUT):
    s = d_ref[...] * (pa_ref[...] + pb_ref[...] + g2_ref[...]) + b2_ref[...]
    s = s[:, :DOUT]
    m = jnp.max(s, axis=1, keepdims=True)
    e = jnp.exp(s - m)
    o_ref[...] = e / jnp.sum(e, axis=1, keepdims=True)


def _fuse2(pa, pb, g2, d, b2, DOUT, R):
    N, DP = g2.shape
    return pl.pallas_call(
        functools.partial(_fuse2_body, DOUT=DOUT),
        grid=(pl.cdiv(N, R),),
        in_specs=[
            pl.BlockSpec((R, DP), lambda i: (i, 0)),
            pl.BlockSpec((R, DP), lambda i: (i, 0)),
            pl.BlockSpec((R, DP), lambda i: (i, 0)),
            pl.BlockSpec((R, 1), lambda i: (i, 0)),
            pl.BlockSpec((1, DP), lambda i: (0, 0)),
        ],
        out_specs=pl.BlockSpec((R, DOUT), lambda i: (i, 0)),
        out_shape=jax.ShapeDtypeStruct((N, DOUT), jnp.float32),
    )(pa, pb, g2, d, b2)


def kernel(x, edge_index, W1, b1, W2, b2):
    N, DIN = x.shape
    DH = W1.shape[1]
    DOUT = W2.shape[1]
    DP = 48
    E = edge_index.shape[1]

    K = 4 * (-(-E // (NW * C * 4)))
    pad = NW * K * C - E

    NP = (NS * 8) * (-(-(N + 1) // (NS * 8)))
    RPT = NP // NS

    dump = N + jnp.arange(pad, dtype=jnp.int32) % jnp.int32(NP - N)
    psrc = jnp.arange(pad, dtype=jnp.int32) % jnp.int32(N)
    src = jnp.concatenate(
        [edge_index[0], psrc]).reshape(NW, K, C)
    dst = jnp.concatenate(
        [edge_index[1], dump]).reshape(NW, K, C)

    degp = _make_deg(K, NP, RPT)(dst, jnp.zeros((RPT, 16), jnp.float32))
    deg = degp[0, :N, 0] + degp[1, :N, 0] + 1.0
    dcol = (deg ** -0.5).reshape(N, 1)

    R = 1000 if N % 1000 == 0 else 8 * (-(-N // 80) // 8 * 8 or 8)

    g1 = _mm_scale(x, W1, dcol, R)
    p1 = _make_agg(DH, K, NP, RPT)(
        g1, src, dst, jnp.zeros((RPT, DH), jnp.float32))

    W2p = jnp.pad(W2, ((0, 0), (0, DP - DOUT)))
    b2p = jnp.pad(b2, (0, DP - DOUT)).reshape(1, DP)
    g2 = _fuse1(p1[0, :N], p1[1, :N], g1, dcol, b1.reshape(1, DH), W2p, R)
    p2 = _make_agg(DP, K, NP, RPT)(
        g2, src, dst, jnp.zeros((RPT, DP), jnp.float32))

    return _fuse2(p2[0, :N], p2[1, :N], g2, dcol, b2p, DOUT, R)


# speedup vs baseline: 3.3340x; 1.1566x over previous
import functools

import jax
import jax.numpy as jnp
from jax import lax
from jax.experimental import pallas as pl
from jax.experimental.pallas import tpu as pltpu
from jax.experimental.pallas import tpu_sc as plsc

NC = 2
NS = 16
NW = NC * NS
C = 128


def _make_deg(K, NP, RPT):
    mesh = plsc.VectorSubcoreMesh(core_axis_name="c", subcore_axis_name="s")

    @functools.partial(
        pl.kernel,
        out_type=jax.ShapeDtypeStruct((NC, NP), jnp.float32),
        mesh=mesh,
        compiler_params=pltpu.CompilerParams(use_tc_tiling_on_sc=False),
        scratch_types=[
            pltpu.VMEM((K, C), jnp.int32),
            pltpu.VMEM((C,), jnp.float32),
            pltpu.VMEM_SHARED((NP,), jnp.float32),
        ],
    )
    def deg(dst_hbm, z_hbm, out_hbm, dst_v, ones_v, acc):
        cid = lax.axis_index("c")
        sid = lax.axis_index("s")
        wid = sid * NC + cid
        pltpu.sync_copy(dst_hbm.at[wid], dst_v)
        for r in range(C // 16):
            ones_v[pl.ds(r * 16, 16)] = jnp.full((16,), 1.0, jnp.float32)
        pltpu.sync_copy(z_hbm, acc.at[pl.ds(sid * RPT, RPT)])
        plsc.subcore_barrier()

        def body(j, carry):
            pltpu.sync_copy(ones_v, acc.at[dst_v.at[j]], add=True)
            return carry

        lax.fori_loop(0, K, body, 0)
        plsc.subcore_barrier()
        pltpu.sync_copy(acc.at[pl.ds(sid * RPT, RPT)],
                        out_hbm.at[cid, pl.ds(sid * RPT, RPT)])

    return deg


def _make_agg(D, K, NP, RPT):
    mesh = plsc.VectorSubcoreMesh(core_axis_name="c", subcore_axis_name="s")

    @functools.partial(
        pl.kernel,
        out_type=jax.ShapeDtypeStruct((NC, NP, D), jnp.float32),
        mesh=mesh,
        compiler_params=pltpu.CompilerParams(use_tc_tiling_on_sc=False),
        scratch_types=[
            pltpu.VMEM((K, C), jnp.int32),
            pltpu.VMEM((K, C), jnp.int32),
            pltpu.VMEM((4, C, D), jnp.float32),
            pltpu.VMEM_SHARED((NP, D), jnp.float32),
            pltpu.SemaphoreType.DMA,
            pltpu.SemaphoreType.DMA,
            pltpu.SemaphoreType.DMA,
            pltpu.SemaphoreType.DMA,
        ],
    )
    def agg(g_hbm, src_hbm, dst_hbm, z_hbm, out_hbm,
            src_v, dst_v, bufs, acc, s0, s1, s2, s3):
        sems = (s0, s1, s2, s3)
        cid = lax.axis_index("c")
        sid = lax.axis_index("s")
        wid = sid * NC + cid
        pltpu.sync_copy(src_hbm.at[wid], src_v)
        pltpu.sync_copy(dst_hbm.at[wid], dst_v)
        pltpu.sync_copy(z_hbm, acc.at[pl.ds(sid * RPT, RPT)])
        for b in range(4):
            pltpu.async_copy(g_hbm.at[src_v.at[b]], bufs.at[b], sems[b])
        plsc.subcore_barrier()

        def body(jj, carry):
            for b in range(4):
                j = 4 * jj + b
                pltpu.make_async_copy(
                    g_hbm.at[src_v.at[j]], bufs.at[b], sems[b]).wait()
                pltpu.sync_copy(bufs.at[b], acc.at[dst_v.at[j]], add=True)
                pltpu.async_copy(
                    g_hbm.at[src_v.at[jnp.minimum(j + 4, K - 1)]],
                    bufs.at[b], sems[b])
            return carry

        lax.fori_loop(0, K // 4, body, 0)
        for b in range(4):
            pltpu.make_async_copy(
                g_hbm.at[src_v.at[0]], bufs.at[b], sems[b]).wait()
        plsc.subcore_barrier()
        pltpu.sync_copy(acc.at[pl.ds(sid * RPT, RPT)],
                        out_hbm.at[cid, pl.ds(sid * RPT, RPT)])

    return agg


def _mm_scale_body(x_ref, w_ref, d_ref, o_ref):
    o_ref[...] = jnp.dot(x_ref[...], w_ref[...],
                         preferred_element_type=jnp.float32) * d_ref[...]


def _mm_scale(x, w, d, R):
    N, DIN = x.shape
    DH = w.shape[1]
    return pl.pallas_call(
        _mm_scale_body,
        grid=(pl.cdiv(N, R),),
        in_specs=[
            pl.BlockSpec((R, DIN), lambda i: (i, 0)),
            pl.BlockSpec((DIN, DH), lambda i: (0, 0)),
            pl.BlockSpec((R, 1), lambda i: (i, 0)),
        ],
        out_specs=pl.BlockSpec((R, DH), lambda i: (i, 0)),
        out_shape=jax.ShapeDtypeStruct((N, DH), jnp.float32),
    )(x, w, d)


def _fuse1_body(pa_ref, pb_ref, g1_ref, d_ref, b1_ref, w2_ref, o_ref):
    s = pa_ref[0] + pb_ref[0] + g1_ref[...]
    h = jnp.maximum(d_ref[...] * s + b1_ref[...], 0.0)
    o_ref[...] = jnp.dot(h, w2_ref[...],
                         preferred_element_type=jnp.float32) * d_ref[...]


def _fuse1(p1, g1, d, b1, w2, R):
    N, DH = g1.shape
    DP = w2.shape[1]
    return pl.pallas_call(
        _fuse1_body,
        grid=(pl.cdiv(N, R),),
        in_specs=[
            pl.BlockSpec((1, R, DH), lambda i: (0, i, 0)),
            pl.BlockSpec((1, R, DH), lambda i: (1, i, 0)),
            pl.BlockSpec((R, DH), lambda i: (i, 0)),
            pl.BlockSpec((R, 1), lambda i: (i, 0)),
            pl.BlockSpec((1, DH), lambda i: (0, 0)),
            pl.BlockSpec((DH, DP), lambda i: (0, 0)),
        ],
        out_specs=pl.BlockSpec((R, DP), lambda i: (i, 0)),
        out_shape=jax.ShapeDtypeStruct((N, DP), jnp.float32),
    )(p1, p1, g1, d, b1, w2)


def _fuse2_body(pa_ref, pb_ref, g2_ref, d_ref, b2_ref, o_ref, *, DOUT):
    s = d_ref[...] * (pa_ref[0] + pb_ref[0] + g2_ref[...]) + b2_ref[...]
    s = s[:, :DOUT]
    m = jnp.max(s, axis=1, keepdims=True)
    e = jnp.exp(s - m)
    o_ref[...] = e / jnp.sum(e, axis=1, keepdims=True)


def _fuse2(p2, g2, d, b2, DOUT, R):
    N, DP = g2.shape
    return pl.pallas_call(
        functools.partial(_fuse2_body, DOUT=DOUT),
        grid=(pl.cdiv(N, R),),
        in_specs=[
            pl.BlockSpec((1, R, DP), lambda i: (0, i, 0)),
            pl.BlockSpec((1, R, DP), lambda i: (1, i, 0)),
            pl.BlockSpec((R, DP), lambda i: (i, 0)),
            pl.BlockSpec((R, 1), lambda i: (i, 0)),
            pl.BlockSpec((1, DP), lambda i: (0, 0)),
        ],
        out_specs=pl.BlockSpec((R, DOUT), lambda i: (i, 0)),
        out_shape=jax.ShapeDtypeStruct((N, DOUT), jnp.float32),
    )(p2, p2, g2, d, b2)


def kernel(x, edge_index, W1, b1, W2, b2):
    N, DIN = x.shape
    DH = W1.shape[1]
    DOUT = W2.shape[1]
    DP = 48
    E = edge_index.shape[1]

    K = 4 * (-(-E // (NW * C * 4)))
    pad = NW * K * C - E

    NP = (NS * 8) * (-(-(N + 1) // (NS * 8)))
    RPT = NP // NS

    dump = N + jnp.arange(pad, dtype=jnp.int32) % jnp.int32(NP - N)
    psrc = jnp.arange(pad, dtype=jnp.int32) % jnp.int32(N)
    src = jnp.concatenate(
        [edge_index[0], psrc]).reshape(NW, K, C)
    dst = jnp.concatenate(
        [edge_index[1], dump]).reshape(NW, K, C)

    degp = _make_deg(K, NP, RPT)(dst, jnp.zeros((RPT,), jnp.float32))
    deg = degp[0, :N] + degp[1, :N] + 1.0
    dcol = (deg ** -0.5).reshape(N, 1)

    R = 2000 if N % 2000 == 0 else 8 * (-(-N // 80) // 8 * 8 or 8)

    g1 = _mm_scale(x, W1, dcol, R)
    p1 = _make_agg(DH, K, NP, RPT)(
        g1, src, dst, jnp.zeros((RPT, DH), jnp.float32))

    W2p = jnp.pad(W2, ((0, 0), (0, DP - DOUT)))
    b2p = jnp.pad(b2, (0, DP - DOUT)).reshape(1, DP)
    g2 = _fuse1(p1, g1, dcol, b1.reshape(1, DH), W2p, R)
    p2 = _make_agg(DP, K, NP, RPT)(
        g2, src, dst, jnp.zeros((RPT, DP), jnp.float32))

    return _fuse2(p2, g2, dcol, b2p, DOUT, R)
